# baseline (speedup 1.0000x reference)
import functools

import jax
import jax.numpy as jnp
from jax import lax
from jax.experimental import pallas as pl
from jax.experimental.pallas import tpu as pltpu
from jax.experimental.pallas import tpu_sc as plsc

MAXDEG = 64
N = 10000
E = 320000
D = 128
NC, NS = 2, 16
HPAD = 10240
SLICE = HPAD // NS
CH = 19968
TAIL = E - CH * NS
BLK = 10000
NBLK = N // BLK
HROW = 10240
UNR = 8


def _hist_body(edges, hist_out, idx_v, tail_v, ones_v, zeros_v, buf_v,
               hist_sh):
    c = lax.axis_index("c")
    s = lax.axis_index("s")

    one16 = jnp.ones((16,), jnp.int32)
    zero16 = jnp.zeros((16,), jnp.int32)

    def obody(i, carry):
        for u in range(UNR):
            ones_v[pl.ds(i * (16 * UNR) + u * 16, 16)] = one16
        return carry

    lax.fori_loop(0, CH // (16 * UNR), obody, 0)

    def zbody(i, carry):
        zeros_v[pl.ds(i * 16, 16)] = zero16
        return carry

    lax.fori_loop(0, SLICE // 16, zbody, 0)
    pltpu.sync_copy(zeros_v, hist_sh.at[pl.ds(s * SLICE, SLICE)])
    pltpu.sync_copy(edges.at[c, pl.ds(s * CH, CH)], idx_v)

    @pl.when(s == 0)
    def _():
        pltpu.sync_copy(edges.at[c, pl.ds(CH * NS, TAIL)], tail_v)

    plsc.subcore_barrier()
    pltpu.sync_copy(ones_v, hist_sh.at[idx_v], add=True)

    @pl.when(s == 0)
    def _():
        pltpu.sync_copy(ones_v.at[pl.ds(0, TAIL)], hist_sh.at[tail_v],
                        add=True)

    plsc.subcore_barrier()

    @pl.when(s < NBLK)
    def _():
        pltpu.sync_copy(hist_sh.at[pl.ds(s * BLK, HROW)], buf_v)
        pltpu.sync_copy(buf_v, hist_out.at[c, s, 0])


_hist_call = pl.kernel(
    _hist_body,
    out_type=jax.ShapeDtypeStruct((NC, NBLK, 1, HROW), jnp.int32),
    mesh=plsc.VectorSubcoreMesh(core_axis_name="c", subcore_axis_name="s",
                                num_cores=NC, num_subcores=NS),
    scratch_types=[
        pltpu.VMEM((CH,), jnp.int32),
        pltpu.VMEM((TAIL,), jnp.int32),
        pltpu.VMEM((CH,), jnp.int32),
        pltpu.VMEM((SLICE,), jnp.int32),
        pltpu.VMEM((HROW,), jnp.int32),
        pltpu.VMEM_SHARED((HPAD,), jnp.int32),
    ],
)


def _encode_body(x_ref, h_ref, ztab_ref, o_ref):
    dout = jnp.minimum(h_ref[0, 0][:, :BLK], MAXDEG - 1)
    din = jnp.minimum(h_ref[1, 0][:, :BLK], MAXDEG - 1)
    iota = lax.broadcasted_iota(jnp.int32, (2 * MAXDEG, BLK), 0)
    sel = (iota == jnp.broadcast_to(dout, (2 * MAXDEG, BLK))) | (
        iota == jnp.broadcast_to(din + MAXDEG, (2 * MAXDEG, BLK)))
    oh = sel.astype(jnp.float32)
    dn = (((0,), (0,)), ((), ()))
    o_ref[...] = x_ref[...] + lax.dot_general(
        oh, ztab_ref[...], dn, preferred_element_type=jnp.float32)


_encode_call = pl.pallas_call(
    _encode_body,
    out_shape=jax.ShapeDtypeStruct((N, D), jnp.float32),
    grid=(NBLK,),
    in_specs=[
        pl.BlockSpec((BLK, D), lambda i: (i, 0)),
        pl.BlockSpec((NC, 1, 1, HROW), lambda i: (0, i, 0, 0)),
        pl.BlockSpec((2 * MAXDEG, D), lambda i: (0, 0)),
    ],
    out_specs=pl.BlockSpec((BLK, D), lambda i: (i, 0)),
    compiler_params=pltpu.CompilerParams(
        dimension_semantics=("parallel",)),
)


def kernel(x, edge_index, z_in, z_out):
    e = edge_index.astype(jnp.int32)
    hist = _hist_call(e)
    ztab = jnp.concatenate([z_out, z_in], axis=0)
    return _encode_call(x, hist, ztab)


# speedup vs baseline: 1.0509x; 1.0509x over previous
import functools

import jax
import jax.numpy as jnp
from jax import lax
from jax.experimental import pallas as pl
from jax.experimental.pallas import tpu as pltpu
from jax.experimental.pallas import tpu_sc as plsc

MAXDEG = 64
N = 10000
E = 320000
D = 128
NC, NS = 2, 16
HPAD = 10240
SLICE = HPAD // NS
CH = 19968
TAIL = E - CH * NS
BLK = 5000
NBLK = N // BLK
HROW = 5120
SUB = HROW * NBLK // NS
UNR = 8


def _hist_body(edges, hist_out, idx_v, tail_v, ones_v, zeros_v, buf_v,
               hist_sh):
    c = lax.axis_index("c")
    s = lax.axis_index("s")

    one16 = jnp.ones((16,), jnp.int32)
    zero16 = jnp.zeros((16,), jnp.int32)

    def obody(i, carry):
        for u in range(UNR):
            ones_v[pl.ds(i * (16 * UNR) + u * 16, 16)] = one16
        return carry

    lax.fori_loop(0, CH // (16 * UNR), obody, 0)

    def zbody(i, carry):
        zeros_v[pl.ds(i * 16, 16)] = zero16
        return carry

    lax.fori_loop(0, SLICE // 16, zbody, 0)
    pltpu.sync_copy(zeros_v, hist_sh.at[pl.ds(s * SLICE, SLICE)])
    pltpu.sync_copy(edges.at[c, pl.ds(s * CH, CH)], idx_v)

    @pl.when(s == 0)
    def _():
        pltpu.sync_copy(edges.at[c, pl.ds(CH * NS, TAIL)], tail_v)

    plsc.subcore_barrier()
    pltpu.sync_copy(ones_v, hist_sh.at[idx_v], add=True)

    @pl.when(s == 0)
    def _():
        pltpu.sync_copy(ones_v.at[pl.ds(0, TAIL)], hist_sh.at[tail_v],
                        add=True)

    plsc.subcore_barrier()

    b = s // (NS // NBLK)
    j = s - b * (NS // NBLK)
    pltpu.sync_copy(hist_sh.at[pl.ds(b * BLK + j * SUB, SUB)], buf_v)
    pltpu.sync_copy(buf_v, hist_out.at[c, b, 0, pl.ds(j * SUB, SUB)])


_hist_call = pl.kernel(
    _hist_body,
    out_type=jax.ShapeDtypeStruct((NC, NBLK, 1, HROW), jnp.int32),
    mesh=plsc.VectorSubcoreMesh(core_axis_name="c", subcore_axis_name="s",
                                num_cores=NC, num_subcores=NS),
    scratch_types=[
        pltpu.VMEM((CH,), jnp.int32),
        pltpu.VMEM((TAIL,), jnp.int32),
        pltpu.VMEM((CH,), jnp.int32),
        pltpu.VMEM((SLICE,), jnp.int32),
        pltpu.VMEM((SUB,), jnp.int32),
        pltpu.VMEM_SHARED((HPAD,), jnp.int32),
    ],
)


def _encode_body(x_ref, h_ref, ztab_ref, o_ref):
    dout = jnp.minimum(h_ref[0, 0][:, :BLK], MAXDEG - 1)
    din = jnp.minimum(h_ref[1, 0][:, :BLK], MAXDEG - 1)
    iota = lax.broadcasted_iota(jnp.int32, (2 * MAXDEG, BLK), 0)
    sel = (iota == jnp.broadcast_to(dout, (2 * MAXDEG, BLK))) | (
        iota == jnp.broadcast_to(din + MAXDEG, (2 * MAXDEG, BLK)))
    oh = sel.astype(jnp.float32)
    dn = (((0,), (0,)), ((), ()))
    o_ref[...] = x_ref[...] + lax.dot_general(
        oh, ztab_ref[...], dn, preferred_element_type=jnp.float32)


_encode_call = pl.pallas_call(
    _encode_body,
    out_shape=jax.ShapeDtypeStruct((N, D), jnp.float32),
    grid=(NBLK,),
    in_specs=[
        pl.BlockSpec((BLK, D), lambda i: (i, 0)),
        pl.BlockSpec((NC, 1, 1, HROW), lambda i: (0, i, 0, 0)),
        pl.BlockSpec((2 * MAXDEG, D), lambda i: (0, 0)),
    ],
    out_specs=pl.BlockSpec((BLK, D), lambda i: (i, 0)),
    compiler_params=pltpu.CompilerParams(
        dimension_semantics=("parallel",)),
)


def kernel(x, edge_index, z_in, z_out):
    e = edge_index.astype(jnp.int32)
    hist = _hist_call(e)
    ztab = jnp.concatenate([z_out, z_in], axis=0)
    return _encode_call(x, hist, ztab)
